# baseline (speedup 1.0000x reference)
import functools

import jax
import jax.numpy as jnp
from jax import lax
from jax.experimental import pallas as pl
from jax.experimental.pallas import tpu as pltpu
from jax.experimental.pallas import tpu_sc as plsc

_W = 2048
_R = 4
_QC = _W // _R
_F32 = jnp.float32
_NC = 2
_NS = 16


def _sc_gather_rows(x_flat, idx):
    nrows, d = idx.shape[0], x_flat.shape[1]
    bpw = nrows // (_NC * _NS)

    @functools.partial(
        pl.kernel,
        mesh=plsc.VectorSubcoreMesh(core_axis_name="c", subcore_axis_name="s"),
        out_type=jax.ShapeDtypeStruct((nrows, d), x_flat.dtype),
        scratch_types=[
            pltpu.VMEM((bpw,), jnp.int32),
            pltpu.VMEM((bpw, d), x_flat.dtype),
            pltpu.SemaphoreType.DMA,
        ],
    )
    def gather(x_hbm, idx_hbm, out_hbm, idx_v, rows_v, sem):
        wid = lax.axis_index("s") * _NC + lax.axis_index("c")
        base = wid * bpw
        pltpu.sync_copy(idx_hbm.at[pl.ds(base, bpw)], idx_v)
        pltpu.async_copy(x_hbm.at[idx_v], rows_v, sem).wait()
        pltpu.sync_copy(rows_v, out_hbm.at[pl.ds(base, bpw)])

    return gather(x_flat, idx)


def _attn_body(xq0_ref, xq1_ref, xq2_ref, xq3_ref, xs_ref, xd_ref,
               wq_ref, wk_ref, wv_ref, out_ref, kd_ref, vd_ref):
    si = pl.program_id(1)
    xs = xs_ref[0]
    wq = wq_ref[...]
    wk = wk_ref[...]
    wv = wv_ref[...]

    k = jnp.dot(xs, wk, preferred_element_type=_F32)
    v = jnp.dot(xs, wv, preferred_element_type=_F32)

    @pl.when(si == 0)
    def _project_dilated():
        xd = xd_ref[0]
        kd_ref[...] = jnp.dot(xd, wk, preferred_element_type=_F32)
        vd_ref[...] = jnp.dot(xd, wv, preferred_element_type=_F32)

    us = []
    for i, xq_ref in enumerate((xq0_ref, xq1_ref, xq2_ref, xq3_ref)):
        qi = jnp.dot(xq_ref[0], wq, preferred_element_type=_F32)
        s = jax.lax.dot_general(qi, k, (((1,), (1,)), ((), ())),
                                preferred_element_type=_F32)
        p = jnp.exp2(s)
        den = jnp.sum(p, axis=1)
        u = jnp.dot(p, v, preferred_element_type=_F32)
        if i == 0:
            sd = jax.lax.dot_general(qi, kd_ref[...], (((1,), (1,)), ((), ())),
                                     preferred_element_type=_F32)
            pd = jnp.exp2(sd)
            den = den + jnp.sum(pd, axis=1)
            u = u + jnp.dot(pd, vd_ref[...], preferred_element_type=_F32)
        us.append(u * (1.0 / den)[:, None])
    out_ref[0] = jnp.stack(us, axis=1).reshape(_W, xs.shape[1])


def kernel(x, Wq, Wk, Wv):
    b, n, c = x.shape
    nseg = n // _W
    nd = n // _R
    x1024 = x.reshape(b, nd, _R * c)
    log2e = 1.4426950408889634
    wq_s = Wq * (log2e / jnp.sqrt(jnp.asarray(c, _F32)))

    idx = jnp.arange(b * nd, dtype=jnp.int32) * _R
    xdg = _sc_gather_rows(x.reshape(b * n, c), idx).reshape(b, nd, c)

    def qspec(i):
        return pl.BlockSpec((1, _QC, c), lambda bi, si, i=i: (bi, si, i))

    out = pl.pallas_call(
        _attn_body,
        grid=(b, nseg),
        in_specs=[
            qspec(0), qspec(1), qspec(2), qspec(3),
            pl.BlockSpec((1, _W, c), lambda bi, si: (bi, si, 0)),
            pl.BlockSpec((1, nd, c), lambda bi, si: (bi, 0, 0)),
            pl.BlockSpec((c, c), lambda bi, si: (0, 0)),
            pl.BlockSpec((c, c), lambda bi, si: (0, 0)),
            pl.BlockSpec((c, c), lambda bi, si: (0, 0)),
        ],
        out_specs=pl.BlockSpec((1, _W, c), lambda bi, si: (bi, si, 0)),
        out_shape=jax.ShapeDtypeStruct((b, n, c), _F32),
        scratch_shapes=[pltpu.VMEM((nd, c), _F32), pltpu.VMEM((nd, c), _F32)],
    )(x1024, x1024, x1024, x1024, x, xdg, wq_s, Wk, Wv)
    return out


# speedup vs baseline: 1.0909x; 1.0909x over previous
import functools

import jax
import jax.numpy as jnp
from jax import lax
from jax.experimental import pallas as pl
from jax.experimental.pallas import tpu as pltpu
from jax.experimental.pallas import tpu_sc as plsc

_W = 2048
_R = 4
_QC = _W // _R
_F32 = jnp.float32
_NC = 2
_NS = 16


def _sc_gather_rows(x_flat, idx):
    nrows, d = idx.shape[0], x_flat.shape[1]
    bpw = nrows // (_NC * _NS)

    @functools.partial(
        pl.kernel,
        mesh=plsc.VectorSubcoreMesh(core_axis_name="c", subcore_axis_name="s"),
        out_type=jax.ShapeDtypeStruct((nrows, d), x_flat.dtype),
        scratch_types=[
            pltpu.VMEM((bpw,), jnp.int32),
            pltpu.VMEM((bpw, d), x_flat.dtype),
            pltpu.SemaphoreType.DMA,
        ],
    )
    def gather(x_hbm, idx_hbm, out_hbm, idx_v, rows_v, sem):
        wid = lax.axis_index("s") * _NC + lax.axis_index("c")
        base = wid * bpw
        pltpu.sync_copy(idx_hbm.at[pl.ds(base, bpw)], idx_v)
        pltpu.async_copy(x_hbm.at[idx_v], rows_v, sem).wait()
        pltpu.sync_copy(rows_v, out_hbm.at[pl.ds(base, bpw)])

    return gather(x_flat, idx)


def _attn_body(xs_ref, xd_ref, wq_ref, wk_ref, wv_ref, out_ref,
               kd_ref, vd_ref):
    si = pl.program_id(1)
    xs = xs_ref[0]
    wq = wq_ref[...]
    wk = wk_ref[...]
    wv = wv_ref[...]

    k = jnp.dot(xs, wk, preferred_element_type=_F32)
    v = jnp.dot(xs, wv, preferred_element_type=_F32)

    @pl.when(si == 0)
    def _project_dilated():
        xd = xd_ref[0]
        kd_ref[...] = jnp.dot(xd, wk, preferred_element_type=_F32)
        vd_ref[...] = jnp.dot(xd, wv, preferred_element_type=_F32)

    xs4 = xs.reshape(_QC, _R, xs.shape[1])
    us = []
    for i in range(_R):
        qi = jnp.dot(xs4[:, i, :], wq, preferred_element_type=_F32)
        s = jax.lax.dot_general(qi, k, (((1,), (1,)), ((), ())),
                                preferred_element_type=_F32)
        p = jnp.exp2(s)
        den = jnp.sum(p, axis=1)
        u = jnp.dot(p, v, preferred_element_type=_F32)
        if i == 0:
            sd = jax.lax.dot_general(qi, kd_ref[...], (((1,), (1,)), ((), ())),
                                     preferred_element_type=_F32)
            pd = jnp.exp2(sd)
            den = den + jnp.sum(pd, axis=1)
            u = u + jnp.dot(pd, vd_ref[...], preferred_element_type=_F32)
        us.append(u * (1.0 / den)[:, None])
    out_ref[0] = jnp.stack(us, axis=1).reshape(_W, xs.shape[1])


def kernel(x, Wq, Wk, Wv):
    b, n, c = x.shape
    nseg = n // _W
    nd = n // _R
    log2e = 1.4426950408889634
    wq_s = Wq * (log2e / jnp.sqrt(jnp.asarray(c, _F32)))

    idx = jnp.arange(b * nd, dtype=jnp.int32) * _R
    xdg = _sc_gather_rows(x.reshape(b * n, c), idx).reshape(b, nd, c)

    out = pl.pallas_call(
        _attn_body,
        grid=(b, nseg),
        in_specs=[
            pl.BlockSpec((1, _W, c), lambda bi, si: (bi, si, 0)),
            pl.BlockSpec((1, nd, c), lambda bi, si: (bi, 0, 0)),
            pl.BlockSpec((c, c), lambda bi, si: (0, 0)),
            pl.BlockSpec((c, c), lambda bi, si: (0, 0)),
            pl.BlockSpec((c, c), lambda bi, si: (0, 0)),
        ],
        out_specs=pl.BlockSpec((1, _W, c), lambda bi, si: (bi, si, 0)),
        out_shape=jax.ShapeDtypeStruct((b, n, c), _F32),
        scratch_shapes=[pltpu.VMEM((nd, c), _F32), pltpu.VMEM((nd, c), _F32)],
    )(x, xdg, wq_s, Wk, Wv)
    return out
